# baseline (speedup 1.0000x reference)
import functools

import jax
import jax.numpy as jnp
from jax import lax
from jax.experimental import pallas as pl
from jax.experimental.pallas import tpu as pltpu
from jax.experimental.pallas import tpu_sc as plsc

VOCAB = 8192
ZC = 32
BETA = 0.25
EPS = 1e-12
NROWS = 8192
BR = 512
NB = NROWS // BR


def _vq_body(z_ref, w_ref, idx_ref, emb_ref, ce_ref, vq_ref, e2_ref):
    pid = pl.program_id(0)

    @pl.when(pid == 0)
    def _():
        w = w_ref[...]
        n = jnp.sqrt(jnp.sum(w * w, axis=1, keepdims=True))
        emb = w / jnp.maximum(n, EPS)
        emb_ref[...] = emb
        e2_ref[...] = jnp.sum(emb * emb, axis=1).reshape(1, VOCAB)
        ce_ref[0, 0] = 0.0
        vq_ref[0, 0] = 0.0

    zb = z_ref[...]
    zn = zb / jnp.maximum(jnp.sqrt(jnp.sum(zb * zb, axis=1, keepdims=True)), EPS)
    z2 = jnp.sum(zn * zn, axis=1, keepdims=True)
    emb = emb_ref[...]
    dots_m2 = lax.dot_general(-2.0 * zn, emb, (((1,), (1,)), ((), ())),
                              preferred_element_type=jnp.float32,
                              precision=lax.Precision.DEFAULT)
    d = (z2 + e2_ref[...]) + dots_m2
    dmin = jnp.min(d, axis=1, keepdims=True)
    ids = lax.broadcasted_iota(jnp.int32, d.shape, 1)
    idx = jnp.min(jnp.where(d == dmin, ids, VOCAB), axis=1)
    idx_ref[:, 0] = idx

    dots_l2 = lax.dot_general(jnp.float32(1.4426950408889634) * zn, emb,
                              (((1,), (1,)), ((), ())),
                              preferred_element_type=jnp.float32,
                              precision=lax.Precision.DEFAULT)
    m = (z2 + 1.0 - dmin) * 0.5
    se = lax.dot_general(jnp.exp2(dots_l2), jnp.ones((VOCAB, 1), jnp.float32),
                         (((1,), (0,)), ((), ())),
                         preferred_element_type=jnp.float32,
                         precision=lax.Precision.DEFAULT)
    ce_ref[0, 0] += jnp.sum(jnp.log(se) - m)
    vq_ref[0, 0] += jnp.sum(dmin)

    @pl.when(pid == NB - 1)
    def _():
        ce_ref[0, 0] = ce_ref[0, 0] * (BETA / NROWS)
        vq_ref[0, 0] = vq_ref[0, 0] / (NROWS * ZC)


def _tc_quantize(z_flat, W):
    return pl.pallas_call(
        _vq_body,
        grid=(NB,),
        in_specs=[
            pl.BlockSpec((BR, ZC), lambda i: (i, 0)),
            pl.BlockSpec((VOCAB, ZC), lambda i: (0, 0)),
        ],
        out_specs=[
            pl.BlockSpec((BR, 1), lambda i: (i, 0)),
            pl.BlockSpec((VOCAB, ZC), lambda i: (0, 0)),
            pl.BlockSpec((1, 1), lambda i: (0, 0), memory_space=pltpu.SMEM),
            pl.BlockSpec((1, 1), lambda i: (0, 0), memory_space=pltpu.SMEM),
        ],
        out_shape=[
            jax.ShapeDtypeStruct((NROWS, 1), jnp.int32),
            jax.ShapeDtypeStruct((VOCAB, ZC), jnp.float32),
            jax.ShapeDtypeStruct((1, 1), jnp.float32),
            jax.ShapeDtypeStruct((1, 1), jnp.float32),
        ],
        scratch_shapes=[pltpu.VMEM((1, VOCAB), jnp.float32)],
    )(z_flat, W)


def _sc_gather(table, idx3):
    info = plsc.get_sparse_core_info()
    nc, ns = info.num_cores, info.num_subcores
    nw = nc * ns
    bpw = NROWS // nw
    per_blk = BR // bpw
    mesh = plsc.VectorSubcoreMesh(core_axis_name="c", subcore_axis_name="s")

    @functools.partial(
        pl.kernel, mesh=mesh,
        out_type=jax.ShapeDtypeStruct((NROWS, ZC), jnp.float32),
        scratch_types=[
            pltpu.VMEM((bpw,), jnp.int32),
            pltpu.VMEM((bpw, ZC), jnp.float32),
            pltpu.SemaphoreType.DMA,
        ],
        compiler_params=pltpu.CompilerParams(use_tc_tiling_on_sc=False),
    )
    def k(table_hbm, idx_hbm, out_hbm, idx_v, rows_v, sem):
        wid = lax.axis_index("s") * nc + lax.axis_index("c")
        pltpu.sync_copy(idx_hbm.at[pl.ds(wid * bpw, bpw)], idx_v)
        pltpu.async_copy(table_hbm.at[idx_v], rows_v, sem).wait()
        pltpu.sync_copy(rows_v, out_hbm.at[pl.ds(wid * bpw, bpw)])

    return k(table, idx3)


def kernel(z, W):
    z_flat = jnp.transpose(z, (0, 2, 3, 1)).reshape(NROWS, ZC)
    idx3, emb, ce_s, vq_s = _tc_quantize(z_flat, W)
    zq_flat = _sc_gather(emb, idx3.reshape(NROWS))
    out = jnp.transpose(zq_flat.reshape(8, 32, 32, ZC), (0, 3, 1, 2))
    return out, vq_s[0, 0], ce_s[0, 0]


# speedup vs baseline: 1.5252x; 1.5252x over previous
import functools

import jax
import jax.numpy as jnp
from jax import lax
from jax.experimental import pallas as pl
from jax.experimental.pallas import tpu as pltpu
from jax.experimental.pallas import tpu_sc as plsc

VOCAB = 8192
ZC = 32
BETA = 0.25
EPS = 1e-12
NROWS = 8192
BR = 512
NB = NROWS // BR


def _vq_body(z_ref, w_ref, idx_ref, emb_ref, ce_ref, vq_ref, e2_ref, iota_ref):
    pid = pl.program_id(0)

    @pl.when(pid == 0)
    def _():
        w = w_ref[...]
        n = jnp.sqrt(jnp.sum(w * w, axis=1, keepdims=True))
        emb = w / jnp.maximum(n, EPS)
        emb_ref[...] = emb
        e2_ref[...] = jnp.sum(emb * emb, axis=1).reshape(1, VOCAB)
        ce_ref[0, 0] = 0.0
        vq_ref[0, 0] = 0.0
        iota_ref[...] = lax.broadcasted_iota(
            jnp.int32, (1, VOCAB), 1).astype(jnp.float32)

    zb = z_ref[...]
    zn = zb / jnp.maximum(jnp.sqrt(jnp.sum(zb * zb, axis=1, keepdims=True)), EPS)
    z2 = jnp.sum(zn * zn, axis=1, keepdims=True)
    emb = emb_ref[...]
    dots_m2 = lax.dot_general(-2.0 * zn, emb, (((1,), (1,)), ((), ())),
                              preferred_element_type=jnp.float32,
                              precision=lax.Precision.DEFAULT)
    d = (z2 + e2_ref[...]) + dots_m2
    dmin = jnp.min(d, axis=1, keepdims=True)
    idx = jnp.min(jnp.where(d == dmin, iota_ref[...], jnp.float32(VOCAB)),
                  axis=1).astype(jnp.int32)
    idx_ref[:, 0] = idx

    dots_l2 = lax.dot_general(jnp.float32(1.4426950408889634) * zn, emb,
                              (((1,), (1,)), ((), ())),
                              preferred_element_type=jnp.float32,
                              precision=lax.Precision.DEFAULT)
    m = (z2 + 1.0 - dmin) * 0.5
    se = jnp.sum(jnp.exp2(dots_l2), axis=1, keepdims=True)
    ce_ref[0, 0] += jnp.sum(jnp.log(se) - m)
    vq_ref[0, 0] += jnp.sum(dmin)

    @pl.when(pid == NB - 1)
    def _():
        ce_ref[0, 0] = ce_ref[0, 0] * (BETA / NROWS)
        vq_ref[0, 0] = vq_ref[0, 0] / (NROWS * ZC)


def _tc_quantize(z_flat, W):
    return pl.pallas_call(
        _vq_body,
        grid=(NB,),
        in_specs=[
            pl.BlockSpec((BR, ZC), lambda i: (i, 0)),
            pl.BlockSpec((VOCAB, ZC), lambda i: (0, 0)),
        ],
        out_specs=[
            pl.BlockSpec((BR, 1), lambda i: (i, 0)),
            pl.BlockSpec((VOCAB, ZC), lambda i: (0, 0)),
            pl.BlockSpec((1, 1), lambda i: (0, 0), memory_space=pltpu.SMEM),
            pl.BlockSpec((1, 1), lambda i: (0, 0), memory_space=pltpu.SMEM),
        ],
        out_shape=[
            jax.ShapeDtypeStruct((NROWS, 1), jnp.int32),
            jax.ShapeDtypeStruct((VOCAB, ZC), jnp.float32),
            jax.ShapeDtypeStruct((1, 1), jnp.float32),
            jax.ShapeDtypeStruct((1, 1), jnp.float32),
        ],
        scratch_shapes=[pltpu.VMEM((1, VOCAB), jnp.float32),
                        pltpu.VMEM((1, VOCAB), jnp.float32)],
    )(z_flat, W)


def _sc_gather(table, idx3):
    info = plsc.get_sparse_core_info()
    nc, ns = info.num_cores, info.num_subcores
    nw = nc * ns
    bpw = NROWS // nw
    per_blk = BR // bpw
    mesh = plsc.VectorSubcoreMesh(core_axis_name="c", subcore_axis_name="s")

    @functools.partial(
        pl.kernel, mesh=mesh,
        out_type=jax.ShapeDtypeStruct((NROWS, ZC), jnp.float32),
        scratch_types=[
            pltpu.VMEM((bpw,), jnp.int32),
            pltpu.VMEM((bpw, ZC), jnp.float32),
            pltpu.SemaphoreType.DMA,
        ],
        compiler_params=pltpu.CompilerParams(use_tc_tiling_on_sc=False),
    )
    def k(table_hbm, idx_hbm, out_hbm, idx_v, rows_v, sem):
        wid = lax.axis_index("s") * nc + lax.axis_index("c")
        pltpu.sync_copy(idx_hbm.at[pl.ds(wid * bpw, bpw)], idx_v)
        pltpu.async_copy(table_hbm.at[idx_v], rows_v, sem).wait()
        pltpu.sync_copy(rows_v, out_hbm.at[pl.ds(wid * bpw, bpw)])

    return k(table, idx3)


def kernel(z, W):
    z_flat = jnp.transpose(z, (0, 2, 3, 1)).reshape(NROWS, ZC)
    idx3, emb, ce_s, vq_s = _tc_quantize(z_flat, W)
    zq_flat = _sc_gather(emb, idx3.reshape(NROWS))
    out = jnp.transpose(zq_flat.reshape(8, 32, 32, ZC), (0, 3, 1, 2))
    return out, vq_s[0, 0], ce_s[0, 0]


# speedup vs baseline: 1.7135x; 1.1235x over previous
import functools

import jax
import jax.numpy as jnp
from jax import lax
from jax.experimental import pallas as pl
from jax.experimental.pallas import tpu as pltpu
from jax.experimental.pallas import tpu_sc as plsc

VOCAB = 8192
ZC = 32
BETA = 0.25
EPS = 1e-12
NROWS = 8192
BR = 512
NB = NROWS // BR


def _vq_body(z_ref, w_ref, idx_ref, emb_ref, ce_ref, vq_ref, e2_ref, iota_ref):
    pid = pl.program_id(0)

    @pl.when(pid == 0)
    def _():
        w = w_ref[...]
        n = jnp.sqrt(jnp.sum(w * w, axis=1, keepdims=True))
        emb = w / jnp.maximum(n, EPS)
        emb_ref[...] = emb
        e2_ref[...] = jnp.sum(emb * emb, axis=1).reshape(1, VOCAB)
        ce_ref[0, 0] = 0.0
        vq_ref[0, 0] = 0.0
        iota_ref[...] = lax.broadcasted_iota(
            jnp.int32, (1, VOCAB), 1).astype(jnp.float32)

    zb = z_ref[...]
    zn = zb / jnp.maximum(jnp.sqrt(jnp.sum(zb * zb, axis=1, keepdims=True)), EPS)
    z2 = jnp.sum(zn * zn, axis=1, keepdims=True)
    emb = emb_ref[...]
    dots_m2 = lax.dot_general(-2.0 * zn, emb, (((1,), (1,)), ((), ())),
                              preferred_element_type=jnp.float32,
                              precision=lax.Precision.DEFAULT)
    d = (z2 + e2_ref[...]) + dots_m2
    dmin = jnp.min(d, axis=1, keepdims=True)
    idx = jnp.min(jnp.where(d == dmin, iota_ref[...], jnp.float32(VOCAB)),
                  axis=1).astype(jnp.int32)
    idx_ref[:, 0] = idx

    dots_l2 = jnp.float32(-1.4426950408889634 / 2.0) * dots_m2
    m = (z2 + 1.0 - dmin) * 0.5
    se = jnp.sum(jnp.exp2(dots_l2), axis=1, keepdims=True)
    ce_ref[0, 0] += jnp.sum(jnp.log(se) - m)
    vq_ref[0, 0] += jnp.sum(dmin)

    @pl.when(pid == NB - 1)
    def _():
        ce_ref[0, 0] = ce_ref[0, 0] * (BETA / NROWS)
        vq_ref[0, 0] = vq_ref[0, 0] / (NROWS * ZC)


def _tc_quantize(z_flat, W):
    return pl.pallas_call(
        _vq_body,
        grid=(NB,),
        in_specs=[
            pl.BlockSpec((BR, ZC), lambda i: (i, 0)),
            pl.BlockSpec((VOCAB, ZC), lambda i: (0, 0)),
        ],
        out_specs=[
            pl.BlockSpec((BR, 1), lambda i: (i, 0)),
            pl.BlockSpec((VOCAB, ZC), lambda i: (0, 0)),
            pl.BlockSpec((1, 1), lambda i: (0, 0), memory_space=pltpu.SMEM),
            pl.BlockSpec((1, 1), lambda i: (0, 0), memory_space=pltpu.SMEM),
        ],
        out_shape=[
            jax.ShapeDtypeStruct((NROWS, 1), jnp.int32),
            jax.ShapeDtypeStruct((VOCAB, ZC), jnp.float32),
            jax.ShapeDtypeStruct((1, 1), jnp.float32),
            jax.ShapeDtypeStruct((1, 1), jnp.float32),
        ],
        scratch_shapes=[pltpu.VMEM((1, VOCAB), jnp.float32),
                        pltpu.VMEM((1, VOCAB), jnp.float32)],
    )(z_flat, W)


def _sc_gather(table, idx3):
    info = plsc.get_sparse_core_info()
    nc, ns = info.num_cores, info.num_subcores
    nw = nc * ns
    bpw = NROWS // nw
    per_blk = BR // bpw
    mesh = plsc.VectorSubcoreMesh(core_axis_name="c", subcore_axis_name="s")

    @functools.partial(
        pl.kernel, mesh=mesh,
        out_type=jax.ShapeDtypeStruct((NROWS, ZC), jnp.float32),
        scratch_types=[
            pltpu.VMEM((bpw,), jnp.int32),
            pltpu.VMEM((bpw, ZC), jnp.float32),
            pltpu.SemaphoreType.DMA,
        ],
        compiler_params=pltpu.CompilerParams(use_tc_tiling_on_sc=False),
    )
    def k(table_hbm, idx_hbm, out_hbm, idx_v, rows_v, sem):
        wid = lax.axis_index("s") * nc + lax.axis_index("c")
        pltpu.sync_copy(idx_hbm.at[pl.ds(wid * bpw, bpw)], idx_v)
        pltpu.async_copy(table_hbm.at[idx_v], rows_v, sem).wait()
        pltpu.sync_copy(rows_v, out_hbm.at[pl.ds(wid * bpw, bpw)])

    return k(table, idx3)


def kernel(z, W):
    z_flat = jnp.transpose(z, (0, 2, 3, 1)).reshape(NROWS, ZC)
    idx3, emb, ce_s, vq_s = _tc_quantize(z_flat, W)
    zq_flat = _sc_gather(emb, idx3.reshape(NROWS))
    out = jnp.transpose(zq_flat.reshape(8, 32, 32, ZC), (0, 3, 1, 2))
    return out, vq_s[0, 0], ce_s[0, 0]


# speedup vs baseline: 1.7158x; 1.0013x over previous
import functools

import jax
import jax.numpy as jnp
from jax import lax
from jax.experimental import pallas as pl
from jax.experimental.pallas import tpu as pltpu
from jax.experimental.pallas import tpu_sc as plsc

VOCAB = 8192
ZC = 32
BETA = 0.25
EPS = 1e-12
NROWS = 8192
BR = 512
NB = NROWS // BR


def _vq_body(z_ref, w_ref, idx_ref, emb_ref, ce_ref, vq_ref, e2_ref, iota_ref):
    pid = pl.program_id(0)

    @pl.when(pid == 0)
    def _():
        w = w_ref[...]
        n = jnp.sqrt(jnp.sum(w * w, axis=1, keepdims=True))
        emb = w / jnp.maximum(n, EPS)
        emb_ref[...] = emb
        e2_ref[...] = jnp.sum(emb * emb, axis=1).reshape(1, VOCAB)
        ce_ref[0, 0] = 0.0
        vq_ref[0, 0] = 0.0
        iota_ref[...] = lax.broadcasted_iota(
            jnp.int32, (1, VOCAB), 1).astype(jnp.float32)

    zb = z_ref[...]
    zn = zb / jnp.maximum(jnp.sqrt(jnp.sum(zb * zb, axis=1, keepdims=True)), EPS)
    z2 = jnp.sum(zn * zn, axis=1, keepdims=True)
    emb = emb_ref[...]
    dots_m2 = lax.dot_general(-2.0 * zn, emb, (((1,), (1,)), ((), ())),
                              preferred_element_type=jnp.float32,
                              precision=lax.Precision.DEFAULT)
    d = (z2 + e2_ref[...]) + dots_m2
    dmin = jnp.min(d, axis=1, keepdims=True)
    idx = jnp.min(jnp.where(d == dmin, iota_ref[...], jnp.float32(VOCAB)),
                  axis=1).astype(jnp.int32)
    idx_ref[:, 0] = idx

    dots_l2 = jnp.float32(-1.4426950408889634 / 2.0) * dots_m2
    m = (z2 + 1.0 - dmin) * 0.5
    se = jnp.sum(jnp.exp2(dots_l2), axis=1, keepdims=True)
    ce_ref[0, 0] += jnp.sum(jnp.log(se) - m)
    vq_ref[0, 0] += jnp.sum(dmin)

    @pl.when(pid == NB - 1)
    def _():
        ce_ref[0, 0] = ce_ref[0, 0] * (BETA / NROWS)
        vq_ref[0, 0] = vq_ref[0, 0] / (NROWS * ZC)


def _tc_quantize(z_flat, W):
    return pl.pallas_call(
        _vq_body,
        grid=(NB,),
        in_specs=[
            pl.BlockSpec((BR, ZC), lambda i: (i, 0)),
            pl.BlockSpec((VOCAB, ZC), lambda i: (0, 0)),
        ],
        out_specs=[
            pl.BlockSpec((BR, 1), lambda i: (i, 0)),
            pl.BlockSpec((VOCAB, ZC), lambda i: (0, 0)),
            pl.BlockSpec((1, 1), lambda i: (0, 0), memory_space=pltpu.SMEM),
            pl.BlockSpec((1, 1), lambda i: (0, 0), memory_space=pltpu.SMEM),
        ],
        out_shape=[
            jax.ShapeDtypeStruct((NROWS, 1), jnp.int32),
            jax.ShapeDtypeStruct((VOCAB, ZC), jnp.float32),
            jax.ShapeDtypeStruct((1, 1), jnp.float32),
            jax.ShapeDtypeStruct((1, 1), jnp.float32),
        ],
        scratch_shapes=[pltpu.VMEM((1, VOCAB), jnp.float32),
                        pltpu.VMEM((1, VOCAB), jnp.float32)],
        compiler_params=pltpu.CompilerParams(
            allow_input_fusion=[True, False]),
    )(z_flat, W)


def _sc_gather(table, idx3):
    info = plsc.get_sparse_core_info()
    nc, ns = info.num_cores, info.num_subcores
    nw = nc * ns
    bpw = NROWS // nw
    per_blk = BR // bpw
    mesh = plsc.VectorSubcoreMesh(core_axis_name="c", subcore_axis_name="s")

    @functools.partial(
        pl.kernel, mesh=mesh,
        out_type=jax.ShapeDtypeStruct((NROWS, ZC), jnp.float32),
        scratch_types=[
            pltpu.VMEM((bpw,), jnp.int32),
            pltpu.VMEM((bpw, ZC), jnp.float32),
            pltpu.SemaphoreType.DMA,
        ],
        compiler_params=pltpu.CompilerParams(use_tc_tiling_on_sc=False),
    )
    def k(table_hbm, idx_hbm, out_hbm, idx_v, rows_v, sem):
        wid = lax.axis_index("s") * nc + lax.axis_index("c")
        pltpu.sync_copy(idx_hbm.at[pl.ds(wid * bpw, bpw)], idx_v)
        pltpu.async_copy(table_hbm.at[idx_v], rows_v, sem).wait()
        pltpu.sync_copy(rows_v, out_hbm.at[pl.ds(wid * bpw, bpw)])

    return k(table, idx3)


def kernel(z, W):
    z_flat = jnp.transpose(z, (0, 2, 3, 1)).reshape(NROWS, ZC)
    idx3, emb, ce_s, vq_s = _tc_quantize(z_flat, W)
    zq_flat = _sc_gather(emb, idx3.reshape(NROWS))
    out = jnp.transpose(zq_flat.reshape(8, 32, 32, ZC), (0, 3, 1, 2))
    return out, vq_s[0, 0], ce_s[0, 0]


# speedup vs baseline: 1.7518x; 1.0210x over previous
import functools

import jax
import jax.numpy as jnp
from jax import lax
from jax.experimental import pallas as pl
from jax.experimental.pallas import tpu as pltpu
from jax.experimental.pallas import tpu_sc as plsc

VOCAB = 8192
ZC = 32
BETA = 0.25
EPS = 1e-12
NROWS = 8192
BR = 1024
NB = NROWS // BR


def _vq_body(z_ref, w_ref, idx_ref, emb_ref, ce_ref, vq_ref, e2_ref, iota_ref):
    pid = pl.program_id(0)

    @pl.when(pid == 0)
    def _():
        w = w_ref[...]
        n = jnp.sqrt(jnp.sum(w * w, axis=1, keepdims=True))
        emb = w / jnp.maximum(n, EPS)
        emb_ref[...] = emb
        e2_ref[...] = jnp.sum(emb * emb, axis=1).reshape(1, VOCAB)
        ce_ref[0, 0] = 0.0
        vq_ref[0, 0] = 0.0
        iota_ref[...] = lax.broadcasted_iota(
            jnp.int32, (1, VOCAB), 1).astype(jnp.float32)

    zb = z_ref[...]
    zn = zb / jnp.maximum(jnp.sqrt(jnp.sum(zb * zb, axis=1, keepdims=True)), EPS)
    z2 = jnp.sum(zn * zn, axis=1, keepdims=True)
    emb = emb_ref[...]
    dots_m2 = lax.dot_general(-2.0 * zn, emb, (((1,), (1,)), ((), ())),
                              preferred_element_type=jnp.float32,
                              precision=lax.Precision.DEFAULT)
    d = (z2 + e2_ref[...]) + dots_m2
    dmin = jnp.min(d, axis=1, keepdims=True)
    idx = jnp.min(jnp.where(d == dmin, iota_ref[...], jnp.float32(VOCAB)),
                  axis=1).astype(jnp.int32)
    idx_ref[:, 0] = idx

    dots_l2 = jnp.float32(-1.4426950408889634 / 2.0) * dots_m2
    m = (z2 + 1.0 - dmin) * 0.5
    se = jnp.sum(jnp.exp2(dots_l2), axis=1, keepdims=True)
    ce_ref[0, 0] += jnp.sum(jnp.log(se) - m)
    vq_ref[0, 0] += jnp.sum(dmin)

    @pl.when(pid == NB - 1)
    def _():
        ce_ref[0, 0] = ce_ref[0, 0] * (BETA / NROWS)
        vq_ref[0, 0] = vq_ref[0, 0] / (NROWS * ZC)


def _tc_quantize(z_flat, W):
    return pl.pallas_call(
        _vq_body,
        grid=(NB,),
        in_specs=[
            pl.BlockSpec((BR, ZC), lambda i: (i, 0)),
            pl.BlockSpec((VOCAB, ZC), lambda i: (0, 0)),
        ],
        out_specs=[
            pl.BlockSpec((BR, 1), lambda i: (i, 0)),
            pl.BlockSpec((VOCAB, ZC), lambda i: (0, 0)),
            pl.BlockSpec((1, 1), lambda i: (0, 0), memory_space=pltpu.SMEM),
            pl.BlockSpec((1, 1), lambda i: (0, 0), memory_space=pltpu.SMEM),
        ],
        out_shape=[
            jax.ShapeDtypeStruct((NROWS, 1), jnp.int32),
            jax.ShapeDtypeStruct((VOCAB, ZC), jnp.float32),
            jax.ShapeDtypeStruct((1, 1), jnp.float32),
            jax.ShapeDtypeStruct((1, 1), jnp.float32),
        ],
        scratch_shapes=[pltpu.VMEM((1, VOCAB), jnp.float32),
                        pltpu.VMEM((1, VOCAB), jnp.float32)],
        compiler_params=pltpu.CompilerParams(
            allow_input_fusion=[True, False],
            vmem_limit_bytes=120 * 1024 * 1024),
    )(z_flat, W)


def _sc_gather(table, idx3):
    info = plsc.get_sparse_core_info()
    nc, ns = info.num_cores, info.num_subcores
    nw = nc * ns
    bpw = NROWS // nw
    per_blk = BR // bpw
    mesh = plsc.VectorSubcoreMesh(core_axis_name="c", subcore_axis_name="s")

    @functools.partial(
        pl.kernel, mesh=mesh,
        out_type=jax.ShapeDtypeStruct((NROWS, ZC), jnp.float32),
        scratch_types=[
            pltpu.VMEM((bpw,), jnp.int32),
            pltpu.VMEM((bpw, ZC), jnp.float32),
            pltpu.SemaphoreType.DMA,
        ],
        compiler_params=pltpu.CompilerParams(use_tc_tiling_on_sc=False),
    )
    def k(table_hbm, idx_hbm, out_hbm, idx_v, rows_v, sem):
        wid = lax.axis_index("s") * nc + lax.axis_index("c")
        pltpu.sync_copy(idx_hbm.at[pl.ds(wid * bpw, bpw)], idx_v)
        pltpu.async_copy(table_hbm.at[idx_v], rows_v, sem).wait()
        pltpu.sync_copy(rows_v, out_hbm.at[pl.ds(wid * bpw, bpw)])

    return k(table, idx3)


def kernel(z, W):
    z_flat = jnp.transpose(z, (0, 2, 3, 1)).reshape(NROWS, ZC)
    idx3, emb, ce_s, vq_s = _tc_quantize(z_flat, W)
    zq_flat = _sc_gather(emb, idx3.reshape(NROWS))
    out = jnp.transpose(zq_flat.reshape(8, 32, 32, ZC), (0, 3, 1, 2))
    return out, vq_s[0, 0], ce_s[0, 0]


# speedup vs baseline: 1.8149x; 1.0360x over previous
import functools

import jax
import jax.numpy as jnp
from jax import lax
from jax.experimental import pallas as pl
from jax.experimental.pallas import tpu as pltpu
from jax.experimental.pallas import tpu_sc as plsc

VOCAB = 8192
ZC = 32
BETA = 0.25
EPS = 1e-12
NROWS = 8192
BR = 1024
NB = NROWS // BR


def _vq_body(z_ref, w_ref, idx_ref, emb_ref, ce_ref, vq_ref, e2_ref, iota_ref):
    pid = pl.program_id(0)

    @pl.when(pid == 0)
    def _():
        w = w_ref[...]
        n = jnp.sqrt(jnp.sum(w * w, axis=1, keepdims=True))
        emb = w / jnp.maximum(n, EPS)
        emb_ref[...] = emb
        e2_ref[...] = jnp.transpose(jnp.sum(emb * emb, axis=1, keepdims=True))
        ce_ref[0, 0] = 0.0
        vq_ref[0, 0] = 0.0
        iota_ref[...] = lax.broadcasted_iota(
            jnp.int32, (1, VOCAB), 1).astype(jnp.float32)

    zb = z_ref[...]
    zn = zb / jnp.maximum(jnp.sqrt(jnp.sum(zb * zb, axis=1, keepdims=True)), EPS)
    z2 = jnp.sum(zn * zn, axis=1, keepdims=True)
    emb = emb_ref[...]
    dots_m2 = lax.dot_general(-2.0 * zn, emb, (((1,), (1,)), ((), ())),
                              preferred_element_type=jnp.float32,
                              precision=lax.Precision.DEFAULT)
    d = (z2 + e2_ref[...]) + dots_m2
    dmin = jnp.min(d, axis=1, keepdims=True)
    idx = jnp.min(jnp.where(d == dmin, iota_ref[...], jnp.float32(VOCAB)),
                  axis=1).astype(jnp.int32)
    idx_ref[:, 0] = idx

    dots_l2 = jnp.float32(-1.4426950408889634 / 2.0) * dots_m2
    m = (z2 + 1.0 - dmin) * 0.5
    se = jnp.sum(jnp.exp2(dots_l2), axis=1, keepdims=True)
    ce_ref[0, 0] += jnp.sum(jnp.log(se) - m)
    vq_ref[0, 0] += jnp.sum(dmin)

    @pl.when(pid == NB - 1)
    def _():
        ce_ref[0, 0] = ce_ref[0, 0] * (BETA / NROWS)
        vq_ref[0, 0] = vq_ref[0, 0] / (NROWS * ZC)


def _tc_quantize(z_flat, W):
    return pl.pallas_call(
        _vq_body,
        grid=(NB,),
        in_specs=[
            pl.BlockSpec((BR, ZC), lambda i: (i, 0)),
            pl.BlockSpec((VOCAB, ZC), lambda i: (0, 0)),
        ],
        out_specs=[
            pl.BlockSpec((BR, 1), lambda i: (i, 0)),
            pl.BlockSpec((VOCAB, ZC), lambda i: (0, 0)),
            pl.BlockSpec((1, 1), lambda i: (0, 0), memory_space=pltpu.SMEM),
            pl.BlockSpec((1, 1), lambda i: (0, 0), memory_space=pltpu.SMEM),
        ],
        out_shape=[
            jax.ShapeDtypeStruct((NROWS, 1), jnp.int32),
            jax.ShapeDtypeStruct((VOCAB, ZC), jnp.float32),
            jax.ShapeDtypeStruct((1, 1), jnp.float32),
            jax.ShapeDtypeStruct((1, 1), jnp.float32),
        ],
        scratch_shapes=[pltpu.VMEM((1, VOCAB), jnp.float32),
                        pltpu.VMEM((1, VOCAB), jnp.float32)],
        compiler_params=pltpu.CompilerParams(
            allow_input_fusion=[True, False],
            vmem_limit_bytes=120 * 1024 * 1024),
    )(z_flat, W)


def _sc_gather(table, idx3):
    info = plsc.get_sparse_core_info()
    nc, ns = info.num_cores, info.num_subcores
    nw = nc * ns
    bpw = NROWS // nw
    per_blk = BR // bpw
    mesh = plsc.VectorSubcoreMesh(core_axis_name="c", subcore_axis_name="s")

    @functools.partial(
        pl.kernel, mesh=mesh,
        out_type=jax.ShapeDtypeStruct((NROWS, ZC), jnp.float32),
        scratch_types=[
            pltpu.VMEM((bpw,), jnp.int32),
            pltpu.VMEM((bpw, ZC), jnp.float32),
            pltpu.SemaphoreType.DMA,
        ],
        compiler_params=pltpu.CompilerParams(use_tc_tiling_on_sc=False),
    )
    def k(table_hbm, idx_hbm, out_hbm, idx_v, rows_v, sem):
        wid = lax.axis_index("s") * nc + lax.axis_index("c")
        pltpu.sync_copy(idx_hbm.at[pl.ds(wid * bpw, bpw)], idx_v)
        pltpu.async_copy(table_hbm.at[idx_v], rows_v, sem).wait()
        pltpu.sync_copy(rows_v, out_hbm.at[pl.ds(wid * bpw, bpw)])

    return k(table, idx3)


def kernel(z, W):
    z_flat = jnp.transpose(z, (0, 2, 3, 1)).reshape(NROWS, ZC)
    idx3, emb, ce_s, vq_s = _tc_quantize(z_flat, W)
    zq_flat = _sc_gather(emb, idx3.reshape(NROWS))
    out = jnp.transpose(zq_flat.reshape(8, 32, 32, ZC), (0, 3, 1, 2))
    return out, vq_s[0, 0], ce_s[0, 0]
